# baseline (speedup 1.0000x reference)
import functools
import jax
import jax.numpy as jnp
from jax import lax
from jax.experimental import pallas as pl
from jax.experimental.pallas import tpu as pltpu
from jax.experimental.pallas import tpu_sc as plsc

N_NODES = 10000
N_EDGES = 320000
H = 128
HEADS = 4
DH = H // HEADS

ROWS = 1024

NC = 2
NS = 16
NSUB = NC * NS
EPW = N_EDGES // NSUB
B_SCAN = 400
G = 128
CAPG = (EPW + G - 1) // G
NPAD = 10240
RPW = N_NODES // NS


def _shuffle16(x, idx):
    dnums = lax.GatherDimensionNumbers(
        offset_dims=(), collapsed_slice_dims=(0,), start_index_map=(0,))
    return lax.gather(x, idx[:, None], dnums, slice_sizes=(1,),
                      mode=lax.GatherScatterMode.PROMISE_IN_BOUNDS)


def _prefix16(x):
    iota = lax.iota(jnp.int32, 16)
    for k in (1, 2, 4, 8):
        sh = _shuffle16(x, jnp.maximum(iota - k, 0))
        x = x + jnp.where(iota >= k, sh, 0)
    return x


def _sc_scatter2_body(wh_ref, w_ref, src_ref, dst_ref,
                      part_ref, svec, dvec, rows4, staged4, wvec, acc_sh,
                      sem):
    c = lax.axis_index("c")
    s = lax.axis_index("s")
    wid = s * NC + c
    zero16 = jnp.zeros((16,), jnp.float32)
    NV = B_SCAN // 16
    NQ = NV // 4

    def zrow(i, _):
        for j in range(H // 16):
            staged4[i, pl.ds(j * 16, 16)] = zero16
        return 0
    lax.fori_loop(0, 64, zrow, 0)

    for t in range(NPAD // G // NS):
        for u in range(G // 16):
            pltpu.sync_copy(staged4.at[pl.ds(0, 16)],
                            acc_sh.at[pl.ds((s + NS * t) * G + u * 16, 16)])
    plsc.subcore_barrier()


    def do_vreg(i, u):
        d = dvec[pl.ds(i * 16, 16)]
        mask = d < N_NODES

        def per_edge(e, _):
            wr = wvec[pl.ds((i * 16 + e) * 16, 16)]
            for j in range(H // 16):
                wsp = _shuffle16(wr, jnp.full((16,), j // 2, jnp.int32))
                staged4[u * 16 + e, pl.ds(j * 16, 16)] = (
                    rows4[u * 16 + e, pl.ds(j * 16, 16)] * wsp)
            return 0
        lax.fori_loop(0, 16, per_edge, 0)
        dm = jnp.where(mask, d, NPAD - 1)
        pltpu.sync_copy(staged4.at[pl.ds(u * 16, 16)], acc_sh.at[dm], add=True)

    def scan_blk(blk, _):
        base = wid * EPW + blk * B_SCAN
        pltpu.sync_copy(src_ref.at[pl.ds(base, B_SCAN)], svec)
        pltpu.sync_copy(dst_ref.at[pl.ds(base, B_SCAN)], dvec)
        pltpu.sync_copy(w_ref.at[pl.ds(base * 16, B_SCAN * 16)], wvec)

        def quad(q, _):
            cps = []
            for u in range(4):
                i = q * 4 + u
                sv = svec[pl.ds(i * 16, 16)]
                cps.append(pltpu.async_copy(
                    wh_ref.at[sv], rows4.at[pl.ds(u * 16, 16)], sem))
            for cp in cps:
                cp.wait()
            for u in range(4):
                do_vreg(q * 4 + u, u)
            return 0
        lax.fori_loop(0, NQ, quad, 0)
        for i in range(NQ * 4, NV):
            sv = svec[pl.ds(i * 16, 16)]
            pltpu.async_copy(wh_ref.at[sv], rows4.at[pl.ds(0, 16)], sem).wait()
            do_vreg(i, 0)
        return 0
    lax.fori_loop(0, EPW // B_SCAN, scan_blk, 0)

    plsc.subcore_barrier()
    for t in range(NPAD // G // NS):
        pltpu.sync_copy(acc_sh.at[pl.ds((s + NS * t) * G, G)],
                        part_ref.at[c, pl.ds((s + NS * t) * G, G)])


def _sc_scatter2(wh2, w2pad, s2, d2):
    mesh = plsc.VectorSubcoreMesh(core_axis_name="c", subcore_axis_name="s")
    f = pl.kernel(
        _sc_scatter2_body,
        out_type=jax.ShapeDtypeStruct((NC, NPAD, H), jnp.float32),
        mesh=mesh,
        scratch_types=[
            pltpu.VMEM((B_SCAN,), jnp.int32),
            pltpu.VMEM((B_SCAN,), jnp.int32),
            pltpu.VMEM((64, H), jnp.float32),
            pltpu.VMEM((64, H), jnp.float32),
            pltpu.VMEM((B_SCAN * 16,), jnp.float32),
            pltpu.VMEM_SHARED((NPAD, H), jnp.float32),
            pltpu.SemaphoreType.DMA,
        ],
    )
    return f(wh2, w2pad, s2, d2)


NB = N_NODES // 1000
EBR = 1000


def _fused1_body(nf_ref, s_ref, d_ref, f_ref, w1_ref, w2_ref, w3_ref, b_ref,
                 gw_ref, p_ref, wh_ref, al_ref):
    pid = pl.program_id(0)

    @pl.when(pid < NB)
    def _():
        wh = jnp.dot(nf_ref[...], gw_ref[...], preferred_element_type=jnp.float32)
        wh_ref[...] = wh
        al_ref[...] = jnp.dot(wh, p_ref[...], preferred_element_type=jnp.float32)

    @pl.when(pid >= NB)
    def _():
        ef = jnp.dot(s_ref[...], w1_ref[...], preferred_element_type=jnp.float32)
        ef += jnp.dot(d_ref[...], w2_ref[...], preferred_element_type=jnp.float32)
        ef += jnp.dot(f_ref[...], w3_ref[...], preferred_element_type=jnp.float32)
        ef = jax.nn.relu(ef + b_ref[...])
        wh = jnp.dot(ef, gw_ref[...], preferred_element_type=jnp.float32)
        wh_ref[...] = wh
        al_ref[...] = jnp.dot(wh, p_ref[...], preferred_element_type=jnp.float32)


def _fused1(nf, src_feat, dst_feat, dfo, w1, w2, w3, b, gw, p):
    m = N_NODES + N_EDGES
    nspec = pl.BlockSpec((EBR, H), lambda i: (jnp.minimum(i, NB - 1), 0))
    espec = pl.BlockSpec((EBR, H), lambda i: (jnp.maximum(i - NB, 0), 0))
    full = pl.BlockSpec((H, H), lambda i: (0, 0))
    return pl.pallas_call(
        _fused1_body,
        grid=(m // EBR,),
        in_specs=[nspec, espec, espec, espec, full, full, full,
                  pl.BlockSpec((1, H), lambda i: (0, 0)),
                  full, pl.BlockSpec((H, 16), lambda i: (0, 0))],
        out_specs=[pl.BlockSpec((EBR, H), lambda i: (i, 0)),
                   pl.BlockSpec((EBR, 16), lambda i: (i, 0))],
        out_shape=[jax.ShapeDtypeStruct((m, H), jnp.float32),
                   jax.ShapeDtypeStruct((m, 16), jnp.float32)],
    )(nf, src_feat, dst_feat, dfo, w1, w2, w3, b, gw, p)


def _fused2_body(nf_ref, x_ref, rz_ref, df_ref, w_ref, ve_ref, p_ref,
                 wh_ref, al_ref, ef_ref, se_ref):
    pid = pl.program_id(0)

    @pl.when(pid < NB)
    def _():
        wh = jnp.dot(nf_ref[...], w_ref[...], preferred_element_type=jnp.float32)
        wh_ref[...] = wh
        al_ref[...] = jnp.dot(wh, p_ref[...], preferred_element_type=jnp.float32)
        ef_ref[...] = jnp.zeros((EBR, H), jnp.float32)
        se_ref[...] = jnp.zeros((EBR, 16), jnp.float32)

    @pl.when(pid >= NB)
    def _():
        ef = jax.nn.relu(x_ref[...].astype(jnp.float32)) * rz_ref[...]
        ef_ref[...] = ef
        wh = jnp.dot(ef, w_ref[...], preferred_element_type=jnp.float32)
        wh_ref[...] = wh
        al_ref[...] = jnp.dot(wh, p_ref[...], preferred_element_type=jnp.float32)
        se_ref[...] = jnp.dot(df_ref[...], ve_ref[...], preferred_element_type=jnp.float32)


def _fused2(nf, x, rz, df, w, ve, p):
    m = N_NODES + N_EDGES
    nspec = pl.BlockSpec((EBR, H), lambda i: (jnp.minimum(i, NB - 1), 0))
    espec = pl.BlockSpec((EBR, H), lambda i: (jnp.maximum(i - NB, 0), 0))
    eout = pl.BlockSpec((EBR, H), lambda i: (jnp.maximum(i - NB, 0), 0))
    eout16 = pl.BlockSpec((EBR, 16), lambda i: (jnp.maximum(i - NB, 0), 0))
    full = pl.BlockSpec((H, H), lambda i: (0, 0))
    return pl.pallas_call(
        _fused2_body,
        grid=(m // EBR,),
        in_specs=[nspec, espec, pl.BlockSpec((1, H), lambda i: (0, 0)), espec,
                  full, pl.BlockSpec((H, 16), lambda i: (0, 0)),
                  pl.BlockSpec((H, 16), lambda i: (0, 0))],
        out_specs=[pl.BlockSpec((EBR, H), lambda i: (i, 0)),
                   pl.BlockSpec((EBR, 16), lambda i: (i, 0)),
                   eout, eout16],
        out_shape=[jax.ShapeDtypeStruct((m, H), jnp.float32),
                   jax.ShapeDtypeStruct((m, 16), jnp.float32),
                   jax.ShapeDtypeStruct((N_EDGES, H), jnp.float32),
                   jax.ShapeDtypeStruct((N_EDGES, 16), jnp.float32)],
    )(nf, x, rz, df, w, ve, p)


def _head_proj(a_src, a_dst, a_edge=None):
    head = jnp.arange(H, dtype=jnp.int32) // DH
    onehot = (head[:, None] == jnp.arange(HEADS, dtype=jnp.int32)[None, :]).astype(jnp.float32)
    cols = [a_src.reshape(-1)[:, None] * onehot, a_dst.reshape(-1)[:, None] * onehot]
    if a_edge is not None:
        cols.append(a_edge.reshape(-1)[:, None] * onehot)
    p = jnp.concatenate(cols, axis=1)
    pad = 16 - p.shape[1]
    return jnp.pad(p, ((0, 0), (0, pad)))


def _softmax_scale(e):
    m = jnp.max(e, axis=0, keepdims=True)
    w = jnp.exp(e - m)
    z = jnp.sum(w, axis=0)
    rz = jnp.repeat(1.0 / z, DH).reshape(1, H)
    return w, rz


def kernel(node_edge_feat, dist_feat_order, dist_feat, srcs, dsts, nids, eids,
           edge_index_e2e, edge_index_e2n, fc_W, fc_b, gat_W, gat_a_src,
           gat_a_dst, sgat_W, sgat_We, sgat_a_src, sgat_a_dst, sgat_a_edge):
    nf = node_edge_feat[:N_NODES]

    src_feat = jnp.take(node_edge_feat, srcs, axis=0)
    dst_feat = jnp.take(node_edge_feat, dsts, axis=0)
    p1 = _head_proj(gat_a_src, gat_a_dst)
    w1, w2, w3 = fc_W[:H], fc_W[H:2 * H], fc_W[2 * H:]
    wh1, al1 = _fused1(nf, src_feat, dst_feat, dist_feat_order, w1, w2, w3,
                       fc_b.reshape(1, H), gat_W, p1)

    s1, d1 = edge_index_e2e[0], edge_index_e2e[1]
    e1 = jax.nn.leaky_relu(al1[s1, :HEADS] + al1[d1, HEADS:2 * HEADS], 0.2)
    watt1, rz1 = _softmax_scale(e1)

    vals1 = (watt1[:, :, None] * jnp.take(wh1, s1, axis=0).reshape(-1, HEADS, DH)).reshape(-1, H)
    acc1 = jnp.zeros((N_NODES + N_EDGES, H), jnp.bfloat16).at[d1].add(
        vals1.astype(jnp.bfloat16))

    ef2_raw = jnp.take(acc1, eids, axis=0)

    p2 = _head_proj(sgat_a_src, sgat_a_dst)
    pe = _head_proj(sgat_a_edge, sgat_a_edge)[:, :HEADS]
    ve = jnp.pad(sgat_We @ pe, ((0, 0), (0, 16 - HEADS)))
    wh2, al2, ef2, se2 = _fused2(nf, ef2_raw, rz1, dist_feat, sgat_W, ve, p2)

    s2, d2 = edge_index_e2n[0], edge_index_e2n[1]
    e2 = jax.nn.leaky_relu(al2[s2, :HEADS] + al2[d2, HEADS:2 * HEADS] + se2[:, :HEADS], 0.2)
    watt2, rz2 = _softmax_scale(e2)

    w2pad = jnp.pad(watt2, ((0, 0), (0, 16 - HEADS))).reshape(-1)
    part = _sc_scatter2(wh2, w2pad, s2, d2)
    out_nodes = jax.nn.relu(part[0, :N_NODES] + part[1, :N_NODES]) * rz2

    return jnp.concatenate([out_nodes, ef2], axis=0)


# speedup vs baseline: 1.0518x; 1.0518x over previous
import functools
import jax
import jax.numpy as jnp
from jax import lax
from jax.experimental import pallas as pl
from jax.experimental.pallas import tpu as pltpu
from jax.experimental.pallas import tpu_sc as plsc

N_NODES = 10000
N_EDGES = 320000
H = 128
HEADS = 4
DH = H // HEADS

ROWS = 1024

NC = 2
NS = 16
NSUB = NC * NS
EPW = N_EDGES // NSUB
B_SCAN = 400
G = 128
CAPG = (EPW + G - 1) // G
NPAD = 10240
RPW = N_NODES // NS


def _shuffle16(x, idx):
    dnums = lax.GatherDimensionNumbers(
        offset_dims=(), collapsed_slice_dims=(0,), start_index_map=(0,))
    return lax.gather(x, idx[:, None], dnums, slice_sizes=(1,),
                      mode=lax.GatherScatterMode.PROMISE_IN_BOUNDS)


def _prefix16(x):
    iota = lax.iota(jnp.int32, 16)
    for k in (1, 2, 4, 8):
        sh = _shuffle16(x, jnp.maximum(iota - k, 0))
        x = x + jnp.where(iota >= k, sh, 0)
    return x


def _sc_scatter2_body(wh_ref, w_ref, src_ref, dst_ref,
                      part_ref, svec, dvec, rows4, staged4, wvec, acc_sh,
                      sem):
    c = lax.axis_index("c")
    s = lax.axis_index("s")
    wid = s * NC + c
    zero16 = jnp.zeros((16,), jnp.float32)
    NV = B_SCAN // 16
    NQ = NV // 4

    def zrow(i, _):
        for j in range(H // 16):
            staged4[i, pl.ds(j * 16, 16)] = zero16
        return 0
    lax.fori_loop(0, 64, zrow, 0)

    for t in range(NPAD // G // NS):
        for u in range(G // 16):
            pltpu.sync_copy(staged4.at[pl.ds(0, 16)],
                            acc_sh.at[pl.ds((s + NS * t) * G + u * 16, 16)])
    plsc.subcore_barrier()


    def do_vreg(i, u):
        d = dvec[pl.ds(i * 16, 16)]
        mask = d < N_NODES

        def per_edge(e, _):
            wr = wvec[pl.ds((i * 16 + e) * 16, 16)]
            for j in range(H // 16):
                wsp = _shuffle16(wr, jnp.full((16,), j // 2, jnp.int32))
                staged4[u * 16 + e, pl.ds(j * 16, 16)] = (
                    rows4[u * 16 + e, pl.ds(j * 16, 16)] * wsp)
            return 0
        lax.fori_loop(0, 16, per_edge, 0)
        dm = jnp.where(mask, d, NPAD - 1)
        pltpu.sync_copy(staged4.at[pl.ds(u * 16, 16)], acc_sh.at[dm], add=True)

    def scan_blk(blk, _):
        base = wid * EPW + blk * B_SCAN
        pltpu.sync_copy(src_ref.at[pl.ds(base, B_SCAN)], svec)
        pltpu.sync_copy(dst_ref.at[pl.ds(base, B_SCAN)], dvec)
        pltpu.sync_copy(w_ref.at[pl.ds(base * 16, B_SCAN * 16)], wvec)

        def quad(q, _):
            cps = []
            for u in range(4):
                i = q * 4 + u
                sv = svec[pl.ds(i * 16, 16)]
                cps.append(pltpu.async_copy(
                    wh_ref.at[sv], rows4.at[pl.ds(u * 16, 16)], sem))
            for cp in cps:
                cp.wait()
            for u in range(4):
                do_vreg(q * 4 + u, u)
            return 0
        lax.fori_loop(0, NQ, quad, 0)
        for i in range(NQ * 4, NV):
            sv = svec[pl.ds(i * 16, 16)]
            pltpu.async_copy(wh_ref.at[sv], rows4.at[pl.ds(0, 16)], sem).wait()
            do_vreg(i, 0)
        return 0
    lax.fori_loop(0, EPW // B_SCAN, scan_blk, 0)

    plsc.subcore_barrier()
    for t in range(NPAD // G // NS):
        pltpu.sync_copy(acc_sh.at[pl.ds((s + NS * t) * G, G)],
                        part_ref.at[c, pl.ds((s + NS * t) * G, G)])


def _sc_scatter2(wh2, w2pad, s2, d2):
    mesh = plsc.VectorSubcoreMesh(core_axis_name="c", subcore_axis_name="s")
    f = pl.kernel(
        _sc_scatter2_body,
        out_type=jax.ShapeDtypeStruct((NC, NPAD, H), jnp.float32),
        mesh=mesh,
        scratch_types=[
            pltpu.VMEM((B_SCAN,), jnp.int32),
            pltpu.VMEM((B_SCAN,), jnp.int32),
            pltpu.VMEM((64, H), jnp.float32),
            pltpu.VMEM((64, H), jnp.float32),
            pltpu.VMEM((B_SCAN * 16,), jnp.float32),
            pltpu.VMEM_SHARED((NPAD, H), jnp.float32),
            pltpu.SemaphoreType.DMA,
        ],
    )
    return f(wh2, w2pad, s2, d2)


def _sc_gather2_body(tab_ref, i1_ref, i2_ref, o1_ref, o2_ref,
                     idx, bufa, bufb, sem):
    c = lax.axis_index("c")
    s = lax.axis_index("s")
    wid = s * NC + c
    base = wid * EPW

    for which in range(2):
        i_ref = i1_ref if which == 0 else i2_ref
        o_ref = o1_ref if which == 0 else o2_ref
        pltpu.sync_copy(i_ref.at[pl.ds(base, EPW)], idx)
        cp = pltpu.async_copy(tab_ref.at[idx.at[pl.ds(0, G)]], bufa, sem)

        def chunk(g, _):
            pltpu.make_async_copy(tab_ref.at[pl.ds(0, G)], bufa, sem).wait()
            pltpu.async_copy(tab_ref.at[idx.at[pl.ds((g + 1) * G, G)]],
                             bufb, sem)
            pltpu.sync_copy(bufa, o_ref.at[pl.ds(base + g * G, G)])
            for j in range(G // 16):
                pass
            return 0
        def pair(q, _):
            g0 = q * 2
            pltpu.make_async_copy(tab_ref.at[pl.ds(0, G)], bufa, sem).wait()
            pltpu.async_copy(tab_ref.at[idx.at[pl.ds((g0 + 1) * G, G)]],
                             bufb, sem)
            pltpu.sync_copy(bufa, o_ref.at[pl.ds(base + g0 * G, G)])
            pltpu.make_async_copy(tab_ref.at[pl.ds(0, G)], bufb, sem).wait()
            pltpu.async_copy(tab_ref.at[idx.at[pl.ds((g0 + 2) * G, G)]],
                             bufa, sem)
            pltpu.sync_copy(bufb, o_ref.at[pl.ds(base + (g0 + 1) * G, G)])
            return 0
        lax.fori_loop(0, 38, pair, 0)
        pltpu.make_async_copy(tab_ref.at[pl.ds(0, G)], bufa, sem).wait()
        pltpu.async_copy(tab_ref.at[idx.at[pl.ds(77 * G, G)]], bufb, sem)
        pltpu.sync_copy(bufa, o_ref.at[pl.ds(base + 76 * G, G)])
        pltpu.make_async_copy(tab_ref.at[pl.ds(0, G)], bufb, sem).wait()
        pltpu.async_copy(tab_ref.at[idx.at[pl.ds(78 * G, 16)]],
                         bufa.at[pl.ds(0, 16)], sem)
        pltpu.sync_copy(bufb, o_ref.at[pl.ds(base + 77 * G, G)])
        pltpu.make_async_copy(tab_ref.at[pl.ds(0, 16)],
                              bufa.at[pl.ds(0, 16)], sem).wait()
        pltpu.sync_copy(bufa.at[pl.ds(0, 16)],
                        o_ref.at[pl.ds(base + 78 * G, 16)])


def _sc_gather2(tab, idx1, idx2):
    mesh = plsc.VectorSubcoreMesh(core_axis_name="c", subcore_axis_name="s")
    f = pl.kernel(
        _sc_gather2_body,
        out_type=[jax.ShapeDtypeStruct((N_EDGES, H), jnp.float32),
                  jax.ShapeDtypeStruct((N_EDGES, H), jnp.float32)],
        mesh=mesh,
        scratch_types=[
            pltpu.VMEM((EPW,), jnp.int32),
            pltpu.VMEM((G, H), jnp.float32),
            pltpu.VMEM((G, H), jnp.float32),
            pltpu.SemaphoreType.DMA,
        ],
    )
    return f(tab, idx1, idx2)


NB = N_NODES // 1000
EBR = 1000


def _fused1_body(nf_ref, s_ref, d_ref, f_ref, w1_ref, w2_ref, w3_ref, b_ref,
                 gw_ref, p_ref, wh_ref, al_ref):
    pid = pl.program_id(0)

    @pl.when(pid < NB)
    def _():
        wh = jnp.dot(nf_ref[...], gw_ref[...], preferred_element_type=jnp.float32)
        wh_ref[...] = wh
        al_ref[...] = jnp.dot(wh, p_ref[...], preferred_element_type=jnp.float32)

    @pl.when(pid >= NB)
    def _():
        ef = jnp.dot(s_ref[...], w1_ref[...], preferred_element_type=jnp.float32)
        ef += jnp.dot(d_ref[...], w2_ref[...], preferred_element_type=jnp.float32)
        ef += jnp.dot(f_ref[...], w3_ref[...], preferred_element_type=jnp.float32)
        ef = jax.nn.relu(ef + b_ref[...])
        wh = jnp.dot(ef, gw_ref[...], preferred_element_type=jnp.float32)
        wh_ref[...] = wh
        al_ref[...] = jnp.dot(wh, p_ref[...], preferred_element_type=jnp.float32)


def _fused1(nf, src_feat, dst_feat, dfo, w1, w2, w3, b, gw, p):
    m = N_NODES + N_EDGES
    nspec = pl.BlockSpec((EBR, H), lambda i: (jnp.minimum(i, NB - 1), 0))
    espec = pl.BlockSpec((EBR, H), lambda i: (jnp.maximum(i - NB, 0), 0))
    full = pl.BlockSpec((H, H), lambda i: (0, 0))
    return pl.pallas_call(
        _fused1_body,
        grid=(m // EBR,),
        in_specs=[nspec, espec, espec, espec, full, full, full,
                  pl.BlockSpec((1, H), lambda i: (0, 0)),
                  full, pl.BlockSpec((H, 16), lambda i: (0, 0))],
        out_specs=[pl.BlockSpec((EBR, H), lambda i: (i, 0)),
                   pl.BlockSpec((EBR, 16), lambda i: (i, 0))],
        out_shape=[jax.ShapeDtypeStruct((m, H), jnp.float32),
                   jax.ShapeDtypeStruct((m, 16), jnp.float32)],
    )(nf, src_feat, dst_feat, dfo, w1, w2, w3, b, gw, p)


def _fused2_body(nf_ref, x_ref, rz_ref, df_ref, w_ref, ve_ref, p_ref,
                 wh_ref, al_ref, ef_ref, se_ref):
    pid = pl.program_id(0)

    @pl.when(pid < NB)
    def _():
        wh = jnp.dot(nf_ref[...], w_ref[...], preferred_element_type=jnp.float32)
        wh_ref[...] = wh
        al_ref[...] = jnp.dot(wh, p_ref[...], preferred_element_type=jnp.float32)
        ef_ref[...] = jnp.zeros((EBR, H), jnp.float32)
        se_ref[...] = jnp.zeros((EBR, 16), jnp.float32)

    @pl.when(pid >= NB)
    def _():
        ef = jax.nn.relu(x_ref[...]) * rz_ref[...]
        ef_ref[...] = ef
        wh = jnp.dot(ef, w_ref[...], preferred_element_type=jnp.float32)
        wh_ref[...] = wh
        al_ref[...] = jnp.dot(wh, p_ref[...], preferred_element_type=jnp.float32)
        se_ref[...] = jnp.dot(df_ref[...], ve_ref[...], preferred_element_type=jnp.float32)


def _fused2(nf, x, rz, df, w, ve, p):
    m = N_NODES + N_EDGES
    nspec = pl.BlockSpec((EBR, H), lambda i: (jnp.minimum(i, NB - 1), 0))
    espec = pl.BlockSpec((EBR, H), lambda i: (jnp.maximum(i - NB, 0), 0))
    eout = pl.BlockSpec((EBR, H), lambda i: (jnp.maximum(i - NB, 0), 0))
    eout16 = pl.BlockSpec((EBR, 16), lambda i: (jnp.maximum(i - NB, 0), 0))
    full = pl.BlockSpec((H, H), lambda i: (0, 0))
    return pl.pallas_call(
        _fused2_body,
        grid=(m // EBR,),
        in_specs=[nspec, espec, pl.BlockSpec((1, H), lambda i: (0, 0)), espec,
                  full, pl.BlockSpec((H, 16), lambda i: (0, 0)),
                  pl.BlockSpec((H, 16), lambda i: (0, 0))],
        out_specs=[pl.BlockSpec((EBR, H), lambda i: (i, 0)),
                   pl.BlockSpec((EBR, 16), lambda i: (i, 0)),
                   eout, eout16],
        out_shape=[jax.ShapeDtypeStruct((m, H), jnp.float32),
                   jax.ShapeDtypeStruct((m, 16), jnp.float32),
                   jax.ShapeDtypeStruct((N_EDGES, H), jnp.float32),
                   jax.ShapeDtypeStruct((N_EDGES, 16), jnp.float32)],
    )(nf, x, rz, df, w, ve, p)


def _head_proj(a_src, a_dst, a_edge=None):
    head = jnp.arange(H, dtype=jnp.int32) // DH
    onehot = (head[:, None] == jnp.arange(HEADS, dtype=jnp.int32)[None, :]).astype(jnp.float32)
    cols = [a_src.reshape(-1)[:, None] * onehot, a_dst.reshape(-1)[:, None] * onehot]
    if a_edge is not None:
        cols.append(a_edge.reshape(-1)[:, None] * onehot)
    p = jnp.concatenate(cols, axis=1)
    pad = 16 - p.shape[1]
    return jnp.pad(p, ((0, 0), (0, pad)))


def _softmax_scale(e):
    m = jnp.max(e, axis=0, keepdims=True)
    w = jnp.exp(e - m)
    z = jnp.sum(w, axis=0)
    rz = jnp.repeat(1.0 / z, DH).reshape(1, H)
    return w, rz


def kernel(node_edge_feat, dist_feat_order, dist_feat, srcs, dsts, nids, eids,
           edge_index_e2e, edge_index_e2n, fc_W, fc_b, gat_W, gat_a_src,
           gat_a_dst, sgat_W, sgat_We, sgat_a_src, sgat_a_dst, sgat_a_edge):
    nf = node_edge_feat[:N_NODES]

    src_feat, dst_feat = _sc_gather2(node_edge_feat, srcs, dsts)
    p1 = _head_proj(gat_a_src, gat_a_dst)
    w1, w2, w3 = fc_W[:H], fc_W[H:2 * H], fc_W[2 * H:]
    wh1, al1 = _fused1(nf, src_feat, dst_feat, dist_feat_order, w1, w2, w3,
                       fc_b.reshape(1, H), gat_W, p1)

    s1, d1 = edge_index_e2e[0], edge_index_e2e[1]
    e1 = jax.nn.leaky_relu(al1[s1, :HEADS] + al1[d1, HEADS:2 * HEADS], 0.2)
    watt1, rz1 = _softmax_scale(e1)

    vals1 = (watt1[:, :, None] * jnp.take(wh1, s1, axis=0).reshape(-1, HEADS, DH)).reshape(-1, H)
    acc1 = jnp.zeros((N_NODES + N_EDGES, H), jnp.float32).at[d1].add(vals1)

    ef2_raw = jnp.take(acc1, eids, axis=0)

    p2 = _head_proj(sgat_a_src, sgat_a_dst)
    pe = _head_proj(sgat_a_edge, sgat_a_edge)[:, :HEADS]
    ve = jnp.pad(sgat_We @ pe, ((0, 0), (0, 16 - HEADS)))
    wh2, al2, ef2, se2 = _fused2(nf, ef2_raw, rz1, dist_feat, sgat_W, ve, p2)

    s2, d2 = edge_index_e2n[0], edge_index_e2n[1]
    e2 = jax.nn.leaky_relu(al2[s2, :HEADS] + al2[d2, HEADS:2 * HEADS] + se2[:, :HEADS], 0.2)
    watt2, rz2 = _softmax_scale(e2)

    w2pad = jnp.pad(watt2, ((0, 0), (0, 16 - HEADS))).reshape(-1)
    part = _sc_scatter2(wh2, w2pad, s2, d2)
    out_nodes = jax.nn.relu(part[0, :N_NODES] + part[1, :N_NODES]) * rz2

    return jnp.concatenate([out_nodes, ef2], axis=0)


# speedup vs baseline: 1.0819x; 1.0286x over previous
import functools
import jax
import jax.numpy as jnp
from jax import lax
from jax.experimental import pallas as pl
from jax.experimental.pallas import tpu as pltpu
from jax.experimental.pallas import tpu_sc as plsc

N_NODES = 10000
N_EDGES = 320000
H = 128
HEADS = 4
DH = H // HEADS

ROWS = 1024

NC = 2
NS = 16
NSUB = NC * NS
EPW = N_EDGES // NSUB
B_SCAN = 400
G = 128
CAPG = (EPW + G - 1) // G
NPAD = 10240
RPW = N_NODES // NS


def _shuffle16(x, idx):
    dnums = lax.GatherDimensionNumbers(
        offset_dims=(), collapsed_slice_dims=(0,), start_index_map=(0,))
    return lax.gather(x, idx[:, None], dnums, slice_sizes=(1,),
                      mode=lax.GatherScatterMode.PROMISE_IN_BOUNDS)


def _prefix16(x):
    iota = lax.iota(jnp.int32, 16)
    for k in (1, 2, 4, 8):
        sh = _shuffle16(x, jnp.maximum(iota - k, 0))
        x = x + jnp.where(iota >= k, sh, 0)
    return x


def _sc_scatter2_body(wh_ref, w_ref, src_ref, dst_ref,
                      part_ref, svec, dvec, rows4, staged4, wvec, acc_sh,
                      sem):
    c = lax.axis_index("c")
    s = lax.axis_index("s")
    wid = s * NC + c
    zero16 = jnp.zeros((16,), jnp.float32)
    NV = B_SCAN // 16
    NQ = NV // 4

    def zrow(i, _):
        for j in range(H // 16):
            staged4[i, pl.ds(j * 16, 16)] = zero16
        return 0
    lax.fori_loop(0, 64, zrow, 0)

    for t in range(NPAD // G // NS):
        for u in range(G // 16):
            pltpu.sync_copy(staged4.at[pl.ds(0, 16)],
                            acc_sh.at[pl.ds((s + NS * t) * G + u * 16, 16)])
    plsc.subcore_barrier()


    def do_vreg(i, u):
        d = dvec[pl.ds(i * 16, 16)]
        mask = d < N_NODES

        def per_edge(e, _):
            wr = wvec[pl.ds((i * 16 + e) * 16, 16)]
            for j in range(H // 16):
                wsp = _shuffle16(wr, jnp.full((16,), j // 2, jnp.int32))
                staged4[u * 16 + e, pl.ds(j * 16, 16)] = (
                    rows4[u * 16 + e, pl.ds(j * 16, 16)] * wsp)
            return 0
        lax.fori_loop(0, 16, per_edge, 0)
        dm = jnp.where(mask, d, NPAD - 1)
        pltpu.sync_copy(staged4.at[pl.ds(u * 16, 16)], acc_sh.at[dm], add=True)

    def scan_blk(blk, _):
        base = wid * EPW + blk * B_SCAN
        pltpu.sync_copy(src_ref.at[pl.ds(base, B_SCAN)], svec)
        pltpu.sync_copy(dst_ref.at[pl.ds(base, B_SCAN)], dvec)
        pltpu.sync_copy(w_ref.at[pl.ds(base * 16, B_SCAN * 16)], wvec)

        def quad(q, _):
            cps = []
            for u in range(4):
                i = q * 4 + u
                sv = svec[pl.ds(i * 16, 16)]
                cps.append(pltpu.async_copy(
                    wh_ref.at[sv], rows4.at[pl.ds(u * 16, 16)], sem))
            for cp in cps:
                cp.wait()
            for u in range(4):
                do_vreg(q * 4 + u, u)
            return 0
        lax.fori_loop(0, NQ, quad, 0)
        for i in range(NQ * 4, NV):
            sv = svec[pl.ds(i * 16, 16)]
            pltpu.async_copy(wh_ref.at[sv], rows4.at[pl.ds(0, 16)], sem).wait()
            do_vreg(i, 0)
        return 0
    lax.fori_loop(0, EPW // B_SCAN, scan_blk, 0)

    plsc.subcore_barrier()
    for t in range(NPAD // G // NS):
        pltpu.sync_copy(acc_sh.at[pl.ds((s + NS * t) * G, G)],
                        part_ref.at[c, pl.ds((s + NS * t) * G, G)])


def _sc_scatter2(wh2, w2pad, s2, d2):
    mesh = plsc.VectorSubcoreMesh(core_axis_name="c", subcore_axis_name="s")
    f = pl.kernel(
        _sc_scatter2_body,
        out_type=jax.ShapeDtypeStruct((NC, NPAD, H), jnp.float32),
        mesh=mesh,
        scratch_types=[
            pltpu.VMEM((B_SCAN,), jnp.int32),
            pltpu.VMEM((B_SCAN,), jnp.int32),
            pltpu.VMEM((64, H), jnp.float32),
            pltpu.VMEM((64, H), jnp.float32),
            pltpu.VMEM((B_SCAN * 16,), jnp.float32),
            pltpu.VMEM_SHARED((NPAD, H), jnp.float32),
            pltpu.SemaphoreType.DMA,
        ],
    )
    return f(wh2, w2pad, s2, d2)


def _sc_gather2_body(tab_ref, i1_ref, i2_ref, o1_ref, o2_ref,
                     idx, bufa, bufb, sem):
    c = lax.axis_index("c")
    s = lax.axis_index("s")
    wid = s * NC + c
    base = wid * EPW

    for which in range(2):
        i_ref = i1_ref if which == 0 else i2_ref
        o_ref = o1_ref if which == 0 else o2_ref
        pltpu.sync_copy(i_ref.at[pl.ds(base, EPW)], idx)
        cp = pltpu.async_copy(tab_ref.at[idx.at[pl.ds(0, G)]], bufa, sem)

        def chunk(g, _):
            pltpu.make_async_copy(tab_ref.at[pl.ds(0, G)], bufa, sem).wait()
            pltpu.async_copy(tab_ref.at[idx.at[pl.ds((g + 1) * G, G)]],
                             bufb, sem)
            pltpu.sync_copy(bufa, o_ref.at[pl.ds(base + g * G, G)])
            for j in range(G // 16):
                pass
            return 0
        def pair(q, _):
            g0 = q * 2
            pltpu.make_async_copy(tab_ref.at[pl.ds(0, G)], bufa, sem).wait()
            pltpu.async_copy(tab_ref.at[idx.at[pl.ds((g0 + 1) * G, G)]],
                             bufb, sem)
            pltpu.sync_copy(bufa, o_ref.at[pl.ds(base + g0 * G, G)])
            pltpu.make_async_copy(tab_ref.at[pl.ds(0, G)], bufb, sem).wait()
            pltpu.async_copy(tab_ref.at[idx.at[pl.ds((g0 + 2) * G, G)]],
                             bufa, sem)
            pltpu.sync_copy(bufb, o_ref.at[pl.ds(base + (g0 + 1) * G, G)])
            return 0
        lax.fori_loop(0, 38, pair, 0)
        pltpu.make_async_copy(tab_ref.at[pl.ds(0, G)], bufa, sem).wait()
        pltpu.async_copy(tab_ref.at[idx.at[pl.ds(77 * G, G)]], bufb, sem)
        pltpu.sync_copy(bufa, o_ref.at[pl.ds(base + 76 * G, G)])
        pltpu.make_async_copy(tab_ref.at[pl.ds(0, G)], bufb, sem).wait()
        pltpu.async_copy(tab_ref.at[idx.at[pl.ds(78 * G, 16)]],
                         bufa.at[pl.ds(0, 16)], sem)
        pltpu.sync_copy(bufb, o_ref.at[pl.ds(base + 77 * G, G)])
        pltpu.make_async_copy(tab_ref.at[pl.ds(0, 16)],
                              bufa.at[pl.ds(0, 16)], sem).wait()
        pltpu.sync_copy(bufa.at[pl.ds(0, 16)],
                        o_ref.at[pl.ds(base + 78 * G, 16)])


def _sc_gather2(tab, idx1, idx2):
    mesh = plsc.VectorSubcoreMesh(core_axis_name="c", subcore_axis_name="s")
    f = pl.kernel(
        _sc_gather2_body,
        out_type=[jax.ShapeDtypeStruct((N_EDGES, H), jnp.float32),
                  jax.ShapeDtypeStruct((N_EDGES, H), jnp.float32)],
        mesh=mesh,
        scratch_types=[
            pltpu.VMEM((EPW,), jnp.int32),
            pltpu.VMEM((G, H), jnp.float32),
            pltpu.VMEM((G, H), jnp.float32),
            pltpu.SemaphoreType.DMA,
        ],
    )
    return f(tab, idx1, idx2)


def _sc_gather1_body(tab_ref, i1_ref, o1_ref, idx, bufa, bufb, sem):
    c = lax.axis_index("c")
    sc = lax.axis_index("s")
    wid = sc * NC + c
    base = wid * EPW
    pltpu.sync_copy(i1_ref.at[pl.ds(base, EPW)], idx)
    pltpu.async_copy(tab_ref.at[idx.at[pl.ds(0, G)]], bufa, sem)

    def pair(q, _):
        g0 = q * 2
        pltpu.make_async_copy(tab_ref.at[pl.ds(0, G)], bufa, sem).wait()
        pltpu.async_copy(tab_ref.at[idx.at[pl.ds((g0 + 1) * G, G)]], bufb, sem)
        pltpu.sync_copy(bufa, o1_ref.at[pl.ds(base + g0 * G, G)])
        pltpu.make_async_copy(tab_ref.at[pl.ds(0, G)], bufb, sem).wait()
        pltpu.async_copy(tab_ref.at[idx.at[pl.ds((g0 + 2) * G, G)]], bufa, sem)
        pltpu.sync_copy(bufb, o1_ref.at[pl.ds(base + (g0 + 1) * G, G)])
        return 0
    lax.fori_loop(0, 38, pair, 0)
    pltpu.make_async_copy(tab_ref.at[pl.ds(0, G)], bufa, sem).wait()
    pltpu.async_copy(tab_ref.at[idx.at[pl.ds(77 * G, G)]], bufb, sem)
    pltpu.sync_copy(bufa, o1_ref.at[pl.ds(base + 76 * G, G)])
    pltpu.make_async_copy(tab_ref.at[pl.ds(0, G)], bufb, sem).wait()
    pltpu.async_copy(tab_ref.at[idx.at[pl.ds(78 * G, 16)]],
                     bufa.at[pl.ds(0, 16)], sem)
    pltpu.sync_copy(bufb, o1_ref.at[pl.ds(base + 77 * G, G)])
    pltpu.make_async_copy(tab_ref.at[pl.ds(0, 16)],
                          bufa.at[pl.ds(0, 16)], sem).wait()
    pltpu.sync_copy(bufa.at[pl.ds(0, 16)],
                    o1_ref.at[pl.ds(base + 78 * G, 16)])


def _sc_gather1(tab, idx1):
    mesh = plsc.VectorSubcoreMesh(core_axis_name="c", subcore_axis_name="s")
    f = pl.kernel(
        _sc_gather1_body,
        out_type=jax.ShapeDtypeStruct((N_EDGES, H), jnp.float32),
        mesh=mesh,
        scratch_types=[
            pltpu.VMEM((EPW,), jnp.int32),
            pltpu.VMEM((G, H), jnp.float32),
            pltpu.VMEM((G, H), jnp.float32),
            pltpu.SemaphoreType.DMA,
        ],
    )
    return f(tab, idx1)


NB = N_NODES // 1000
EBR = 1000


def _fused1_body(nf_ref, s_ref, d_ref, f_ref, w1_ref, w2_ref, w3_ref, b_ref,
                 gw_ref, p_ref, wh_ref, al_ref):
    pid = pl.program_id(0)

    @pl.when(pid < NB)
    def _():
        wh = jnp.dot(nf_ref[...], gw_ref[...], preferred_element_type=jnp.float32)
        wh_ref[...] = wh
        al_ref[...] = jnp.dot(wh, p_ref[...], preferred_element_type=jnp.float32)

    @pl.when(pid >= NB)
    def _():
        ef = jnp.dot(s_ref[...], w1_ref[...], preferred_element_type=jnp.float32)
        ef += jnp.dot(d_ref[...], w2_ref[...], preferred_element_type=jnp.float32)
        ef += jnp.dot(f_ref[...], w3_ref[...], preferred_element_type=jnp.float32)
        ef = jax.nn.relu(ef + b_ref[...])
        wh = jnp.dot(ef, gw_ref[...], preferred_element_type=jnp.float32)
        wh_ref[...] = wh
        al_ref[...] = jnp.dot(wh, p_ref[...], preferred_element_type=jnp.float32)


def _fused1(nf, src_feat, dst_feat, dfo, w1, w2, w3, b, gw, p):
    m = N_NODES + N_EDGES
    nspec = pl.BlockSpec((EBR, H), lambda i: (jnp.minimum(i, NB - 1), 0))
    espec = pl.BlockSpec((EBR, H), lambda i: (jnp.maximum(i - NB, 0), 0))
    full = pl.BlockSpec((H, H), lambda i: (0, 0))
    return pl.pallas_call(
        _fused1_body,
        grid=(m // EBR,),
        in_specs=[nspec, espec, espec, espec, full, full, full,
                  pl.BlockSpec((1, H), lambda i: (0, 0)),
                  full, pl.BlockSpec((H, 16), lambda i: (0, 0))],
        out_specs=[pl.BlockSpec((EBR, H), lambda i: (i, 0)),
                   pl.BlockSpec((EBR, 16), lambda i: (i, 0))],
        out_shape=[jax.ShapeDtypeStruct((m, H), jnp.float32),
                   jax.ShapeDtypeStruct((m, 16), jnp.float32)],
    )(nf, src_feat, dst_feat, dfo, w1, w2, w3, b, gw, p)


def _fused2_body(nf_ref, x_ref, rz_ref, df_ref, w_ref, ve_ref, p_ref,
                 wh_ref, al_ref, ef_ref, se_ref):
    pid = pl.program_id(0)

    @pl.when(pid < NB)
    def _():
        wh = jnp.dot(nf_ref[...], w_ref[...], preferred_element_type=jnp.float32)
        wh_ref[...] = wh
        al_ref[...] = jnp.dot(wh, p_ref[...], preferred_element_type=jnp.float32)
        ef_ref[...] = jnp.zeros((EBR, H), jnp.float32)
        se_ref[...] = jnp.zeros((EBR, 16), jnp.float32)

    @pl.when(pid >= NB)
    def _():
        ef = jax.nn.relu(x_ref[...]) * rz_ref[...]
        ef_ref[...] = ef
        wh = jnp.dot(ef, w_ref[...], preferred_element_type=jnp.float32)
        wh_ref[...] = wh
        al_ref[...] = jnp.dot(wh, p_ref[...], preferred_element_type=jnp.float32)
        se_ref[...] = jnp.dot(df_ref[...], ve_ref[...], preferred_element_type=jnp.float32)


def _fused2(nf, x, rz, df, w, ve, p):
    m = N_NODES + N_EDGES
    nspec = pl.BlockSpec((EBR, H), lambda i: (jnp.minimum(i, NB - 1), 0))
    espec = pl.BlockSpec((EBR, H), lambda i: (jnp.maximum(i - NB, 0), 0))
    eout = pl.BlockSpec((EBR, H), lambda i: (jnp.maximum(i - NB, 0), 0))
    eout16 = pl.BlockSpec((EBR, 16), lambda i: (jnp.maximum(i - NB, 0), 0))
    full = pl.BlockSpec((H, H), lambda i: (0, 0))
    return pl.pallas_call(
        _fused2_body,
        grid=(m // EBR,),
        in_specs=[nspec, espec, pl.BlockSpec((1, H), lambda i: (0, 0)), espec,
                  full, pl.BlockSpec((H, 16), lambda i: (0, 0)),
                  pl.BlockSpec((H, 16), lambda i: (0, 0))],
        out_specs=[pl.BlockSpec((EBR, H), lambda i: (i, 0)),
                   pl.BlockSpec((EBR, 16), lambda i: (i, 0)),
                   eout, eout16],
        out_shape=[jax.ShapeDtypeStruct((m, H), jnp.float32),
                   jax.ShapeDtypeStruct((m, 16), jnp.float32),
                   jax.ShapeDtypeStruct((N_EDGES, H), jnp.float32),
                   jax.ShapeDtypeStruct((N_EDGES, 16), jnp.float32)],
    )(nf, x, rz, df, w, ve, p)


def _head_proj(a_src, a_dst, a_edge=None):
    head = jnp.arange(H, dtype=jnp.int32) // DH
    onehot = (head[:, None] == jnp.arange(HEADS, dtype=jnp.int32)[None, :]).astype(jnp.float32)
    cols = [a_src.reshape(-1)[:, None] * onehot, a_dst.reshape(-1)[:, None] * onehot]
    if a_edge is not None:
        cols.append(a_edge.reshape(-1)[:, None] * onehot)
    p = jnp.concatenate(cols, axis=1)
    pad = 16 - p.shape[1]
    return jnp.pad(p, ((0, 0), (0, pad)))


def _softmax_scale(e):
    m = jnp.max(e, axis=0, keepdims=True)
    w = jnp.exp(e - m)
    z = jnp.sum(w, axis=0)
    rz = jnp.repeat(1.0 / z, DH).reshape(1, H)
    return w, rz


def kernel(node_edge_feat, dist_feat_order, dist_feat, srcs, dsts, nids, eids,
           edge_index_e2e, edge_index_e2n, fc_W, fc_b, gat_W, gat_a_src,
           gat_a_dst, sgat_W, sgat_We, sgat_a_src, sgat_a_dst, sgat_a_edge):
    nf = node_edge_feat[:N_NODES]

    src_feat, dst_feat = _sc_gather2(node_edge_feat, srcs, dsts)
    p1 = _head_proj(gat_a_src, gat_a_dst)
    w1, w2, w3 = fc_W[:H], fc_W[H:2 * H], fc_W[2 * H:]
    wh1, al1 = _fused1(nf, src_feat, dst_feat, dist_feat_order, w1, w2, w3,
                       fc_b.reshape(1, H), gat_W, p1)

    s1, d1 = edge_index_e2e[0], edge_index_e2e[1]
    e1 = jax.nn.leaky_relu(al1[s1, :HEADS] + al1[d1, HEADS:2 * HEADS], 0.2)
    watt1, rz1 = _softmax_scale(e1)

    vals1 = (watt1[:, :, None] * _sc_gather1(wh1, s1).reshape(-1, HEADS, DH)).reshape(-1, H)
    acc1 = jnp.zeros((N_NODES + N_EDGES, H), jnp.float32).at[d1].add(vals1)

    ef2_raw = _sc_gather1(acc1, eids)

    p2 = _head_proj(sgat_a_src, sgat_a_dst)
    pe = _head_proj(sgat_a_edge, sgat_a_edge)[:, :HEADS]
    ve = jnp.pad(sgat_We @ pe, ((0, 0), (0, 16 - HEADS)))
    wh2, al2, ef2, se2 = _fused2(nf, ef2_raw, rz1, dist_feat, sgat_W, ve, p2)

    s2, d2 = edge_index_e2n[0], edge_index_e2n[1]
    e2 = jax.nn.leaky_relu(al2[s2, :HEADS] + al2[d2, HEADS:2 * HEADS] + se2[:, :HEADS], 0.2)
    watt2, rz2 = _softmax_scale(e2)

    w2pad = jnp.pad(watt2, ((0, 0), (0, 16 - HEADS))).reshape(-1)
    part = _sc_scatter2(wh2, w2pad, s2, d2)
    out_nodes = jax.nn.relu(part[0, :N_NODES] + part[1, :N_NODES]) * rz2

    return jnp.concatenate([out_nodes, ef2], axis=0)
